# baseline (speedup 1.0000x reference)
import jax
import jax.numpy as jnp
from jax import lax
from jax.experimental import pallas as pl
from jax.experimental.pallas import tpu as pltpu
from jax.experimental.pallas import tpu_sc as plsc

_NUM_CORES = 2
_NUM_SUBCORES = 16
_NW = _NUM_CORES * _NUM_SUBCORES


def _copy_body(w_hbm, out_hbm, sem):
    rows = out_hbm.shape[0]
    rpw = rows // _NW
    wid = lax.axis_index("s") * _NUM_CORES + lax.axis_index("c")
    base = wid * rpw
    pltpu.async_copy(
        w_hbm.at[pl.ds(base, rpw)], out_hbm.at[pl.ds(base, rpw)], sem
    ).wait()


def kernel(x, W):
    seq_len = x.shape[1]
    dim = W.shape[1]
    mesh = plsc.VectorSubcoreMesh(core_axis_name="c", subcore_axis_name="s")
    k = pl.kernel(
        _copy_body,
        out_type=jax.ShapeDtypeStruct((seq_len, dim), W.dtype),
        mesh=mesh,
        scratch_types=[pltpu.SemaphoreType.DMA],
    )
    return k(W)


# speedup vs baseline: 23.7051x; 23.7051x over previous
import jax
import jax.numpy as jnp
from jax import lax
from jax.experimental import pallas as pl
from jax.experimental.pallas import tpu as pltpu
from jax.experimental.pallas import tpu_sc as plsc

_NUM_CORES = 2
_NW = 32
_CHUNK = 32
_NBUF = 3
_LA = 2


def _sc_body(w_hbm, out_hbm, buf, g0, g1, g2, s0, s1, s2):
    gsems = (g0, g1, g2)
    ssems = (s0, s1, s2)
    rows = out_hbm.shape[0]
    rpw = rows // _NW
    nchunks = rpw // _CHUNK
    wid = lax.axis_index("s") * _NUM_CORES + lax.axis_index("c")
    base = wid * rpw

    def gather(j):
        k = j % _NBUF
        return pltpu.async_copy(
            w_hbm.at[pl.ds(base + j * _CHUNK, _CHUNK)], buf.at[k], gsems[k]
        )

    def scatter(i):
        k = i % _NBUF
        return pltpu.async_copy(
            buf.at[k], out_hbm.at[pl.ds(base + i * _CHUNK, _CHUNK)], ssems[k]
        )

    gd, sd = {}, {}
    for j in range(_LA):
        gd[j] = gather(j)
    for i in range(nchunks):
        gd[i].wait()
        sd[i] = scatter(i)
        j = i + _LA
        if j < nchunks:
            if j >= _NBUF:
                sd[j - _NBUF].wait()
            gd[j] = gather(j)
    for i in range(max(0, nchunks - _NBUF), nchunks):
        sd[i].wait()


def kernel(x, W):
    seq_len = x.shape[1]
    dim = W.shape[1]
    mesh = plsc.VectorSubcoreMesh(core_axis_name="c", subcore_axis_name="s")
    k = pl.kernel(
        _sc_body,
        out_type=jax.ShapeDtypeStruct((seq_len, dim), W.dtype),
        mesh=mesh,
        scratch_types=[pltpu.VMEM((_NBUF, _CHUNK, dim), jnp.float32)]
        + [pltpu.SemaphoreType.DMA] * 6,
    )
    return k(W)
